# baseline (speedup 1.0000x reference)
import functools

import jax
import jax.numpy as jnp
from jax import lax
from jax.experimental import pallas as pl
from jax.experimental.pallas import tpu as pltpu
from jax.experimental.pallas import tpu_sc as plsc

NW = 32
C = 128
NB = 4


@functools.partial(jax.jit, static_argnums=(2, 3))
def _sc_embedding(idx, weight, n_chunks, d):
    mesh = plsc.VectorSubcoreMesh(core_axis_name="c", subcore_axis_name="s")

    @functools.partial(
        pl.kernel,
        mesh=mesh,
        out_type=jax.ShapeDtypeStruct((NW * n_chunks, C, d), jnp.float32),
        scratch_types=[
            pltpu.VMEM((n_chunks, C), jnp.int32),
            pltpu.VMEM((NB, C, d), jnp.float32),
        ]
        + [pltpu.SemaphoreType.DMA] * (2 * NB),
        compiler_params=pltpu.CompilerParams(use_tc_tiling_on_sc=False),
    )
    def emb(idx_hbm, table_hbm, out_hbm, idx_v, rows_v, *sems):
        sem_g = sems[:NB]
        sem_s = sems[NB:]
        wid = lax.axis_index("s") * 2 + lax.axis_index("c")
        base = wid * n_chunks

        pltpu.sync_copy(idx_hbm.at[pl.ds(base, n_chunks)], idx_v)

        def g_start(b, i):
            pltpu.async_copy(table_hbm.at[idx_v.at[i]], rows_v.at[b], sem_g[b])

        def g_wait(b, i):
            pltpu.make_async_copy(
                table_hbm.at[idx_v.at[i]], rows_v.at[b], sem_g[b]
            ).wait()

        def s_start(b, i):
            pltpu.async_copy(rows_v.at[b], out_hbm.at[base + i], sem_s[b])

        def s_wait(b, i):
            pltpu.make_async_copy(
                rows_v.at[b], out_hbm.at[base + i], sem_s[b]
            ).wait()

        for b in range(NB):
            g_start(b, b)

        def outer(j, carry):
            i0 = j * NB
            for b in range(NB):
                g_wait(b, i0 + b)
                s_start(b, i0 + b)
            for b in range(NB):
                s_wait(b, i0 + b)
                g_start(b, i0 + NB + b)
            return carry

        lax.fori_loop(0, n_chunks // NB - 1, outer, 0)

        i0 = n_chunks - NB
        for b in range(NB):
            g_wait(b, i0 + b)
            s_start(b, i0 + b)
        for b in range(NB):
            s_wait(b, i0 + b)

    return emb(idx, weight)


def kernel(x, weight):
    b0, b1 = x.shape
    v, d = weight.shape
    n = b0 * b1
    assert n % (NW * C) == 0
    n_chunks = n // (NW * C)
    assert n_chunks % NB == 0
    idx = x.reshape(NW * n_chunks, C).astype(jnp.int32)
    out = _sc_embedding(idx, weight, n_chunks, d)
    return out.reshape(b0, b1, d)
